# Initial kernel scaffold; baseline (speedup 1.0000x reference)
import functools

import jax
import jax.numpy as jnp
from jax import lax
from jax.experimental import pallas as pl
from jax.experimental.pallas import tpu as pltpu
from jax.experimental.pallas import tpu_sc as plsc

N = 10000
D = 128
NC, NS, L = 2, 16, 16
NW = NC * NS
K = 128
NPAD = 10240
RPS = NPAD // NS
DUMMY = N
BM = 1024
GRID = NPAD // BM


def _sc_mesh():
    return plsc.VectorSubcoreMesh(
        core_axis_name="c", subcore_axis_name="s",
        num_cores=NC, num_subcores=NS)


def _deg_partials(src3, dst3, ones_l, zeros_l, cpt):
    @functools.partial(
        pl.kernel,
        out_type=(jax.ShapeDtypeStruct((NC, NPAD, L), jnp.float32),
                  jax.ShapeDtypeStruct((NC, NPAD, L), jnp.float32)),
        mesh=_sc_mesh(),
        scratch_types=[
            pltpu.VMEM((cpt, K), jnp.int32),
            pltpu.VMEM((cpt, K), jnp.int32),
            pltpu.VMEM((K, L), jnp.float32),
            pltpu.VMEM_SHARED((NPAD, L), jnp.float32),
            pltpu.VMEM_SHARED((NPAD, L), jnp.float32),
        ],
    )
    def deg_k(src_hbm, dst_hbm, ones_hbm, z_hbm, dout_hbm, din_hbm,
              src_v, dst_v, ones_v, acc_o, acc_i):
        cid = lax.axis_index("c")
        sid = lax.axis_index("s")
        wid = cid * NS + sid
        pltpu.sync_copy(src_hbm.at[wid], src_v)
        pltpu.sync_copy(dst_hbm.at[wid], dst_v)
        pltpu.sync_copy(ones_hbm, ones_v)
        r0 = sid * RPS
        pltpu.sync_copy(z_hbm.at[pl.ds(r0, RPS)], acc_o.at[pl.ds(r0, RPS)])
        pltpu.sync_copy(z_hbm.at[pl.ds(r0, RPS)], acc_i.at[pl.ds(r0, RPS)])
        plsc.subcore_barrier()

        def body(c, carry):
            pltpu.sync_copy(ones_v, acc_o.at[src_v.at[c]], add=True)
            pltpu.sync_copy(ones_v, acc_i.at[dst_v.at[c]], add=True)
            return carry

        lax.fori_loop(0, cpt, body, 0)
        plsc.subcore_barrier()
        pltpu.sync_copy(acc_o.at[pl.ds(r0, RPS)],
                        dout_hbm.at[cid, pl.ds(r0, RPS)])
        pltpu.sync_copy(acc_i.at[pl.ds(r0, RPS)],
                        din_hbm.at[cid, pl.ds(r0, RPS)])

    return deg_k(src3, dst3, ones_l, zeros_l)


def _agg_partials(h_pad, src3, dst3, zeros_d, cpt):
    @functools.partial(
        pl.kernel,
        out_type=jax.ShapeDtypeStruct((NC, NPAD, D), jnp.float32),
        mesh=_sc_mesh(),
        scratch_types=[
            pltpu.VMEM((cpt, K), jnp.int32),
            pltpu.VMEM((cpt, K), jnp.int32),
            pltpu.VMEM((K, D), jnp.float32),
            pltpu.VMEM_SHARED((NPAD, D), jnp.float32),
            pltpu.SemaphoreType.DMA,
        ],
    )
    def agg_k(h_hbm, src_hbm, dst_hbm, z_hbm, out_hbm,
              src_v, dst_v, rows_v, acc, sem):
        cid = lax.axis_index("c")
        sid = lax.axis_index("s")
        wid = cid * NS + sid
        pltpu.sync_copy(src_hbm.at[wid], src_v)
        pltpu.sync_copy(dst_hbm.at[wid], dst_v)
        r0 = sid * RPS
        pltpu.sync_copy(z_hbm.at[pl.ds(r0, RPS)], acc.at[pl.ds(r0, RPS)])
        plsc.subcore_barrier()

        def body(c, carry):
            pltpu.async_copy(h_hbm.at[src_v.at[c]], rows_v, sem).wait()
            pltpu.sync_copy(rows_v, acc.at[dst_v.at[c]], add=True)
            return carry

        lax.fori_loop(0, cpt, body, 0)
        plsc.subcore_barrier()
        pltpu.sync_copy(acc.at[pl.ds(r0, RPS)],
                        out_hbm.at[cid, pl.ds(r0, RPS)])

    return agg_k(h_pad, src3, dst3, zeros_d)


def _norm_col(pdeg):
    deg = (pdeg[0] + pdeg[1])[:, :1]
    return jnp.where(deg > 0, lax.rsqrt(deg), 0.0)


def _tc1_body(x_ref, w_ref, b_ref, po_ref, o_ref):
    ns = _norm_col(po_ref[...])
    h = jnp.dot(x_ref[...], w_ref[...], preferred_element_type=jnp.float32)
    o_ref[...] = (h + b_ref[...]) * ns


def _tc1(xp, w_fc, b_fc, dout_p):
    return pl.pallas_call(
        _tc1_body,
        grid=(GRID,),
        in_specs=[pl.BlockSpec((BM, D), lambda i: (i, 0)),
                  pl.BlockSpec((D, D), lambda i: (0, 0)),
                  pl.BlockSpec((1, D), lambda i: (0, 0)),
                  pl.BlockSpec((NC, BM, L), lambda i: (0, i, 0))],
        out_specs=pl.BlockSpec((BM, D), lambda i: (i, 0)),
        out_shape=jax.ShapeDtypeStruct((NPAD, D), jnp.float32),
    )(xp, w_fc, b_fc.reshape(1, D), dout_p)


def _tc2_body(a_ref, po_ref, pi_ref, b_ref, o_ref):
    agg = a_ref[0] + a_ref[1]
    nd = _norm_col(pi_ref[...])
    ns = _norm_col(po_ref[...])
    h = jnp.maximum(agg * nd + b_ref[...], 0.0)
    o_ref[...] = h * ns


def _tc2(p1, dout_p, din_p, b_gc1):
    return pl.pallas_call(
        _tc2_body,
        grid=(GRID,),
        in_specs=[pl.BlockSpec((NC, BM, D), lambda i: (0, i, 0)),
                  pl.BlockSpec((NC, BM, L), lambda i: (0, i, 0)),
                  pl.BlockSpec((NC, BM, L), lambda i: (0, i, 0)),
                  pl.BlockSpec((1, D), lambda i: (0, 0))],
        out_specs=pl.BlockSpec((BM, D), lambda i: (i, 0)),
        out_shape=jax.ShapeDtypeStruct((NPAD, D), jnp.float32),
    )(p1, dout_p, din_p, b_gc1.reshape(1, D))


def _tc3_body(a_ref, pi_ref, w_ref, b_ref, o_ref):
    agg = (a_ref[0] + a_ref[1]) * _norm_col(pi_ref[...])
    h = jnp.dot(agg, w_ref[...], preferred_element_type=jnp.float32)
    o_ref[...] = jnp.maximum(h + b_ref[...], 0.0)


def _tc3(p2, din_p, w_gc2, b_gc2):
    return pl.pallas_call(
        _tc3_body,
        grid=(GRID,),
        in_specs=[pl.BlockSpec((NC, BM, D), lambda i: (0, i, 0)),
                  pl.BlockSpec((NC, BM, L), lambda i: (0, i, 0)),
                  pl.BlockSpec((D, D), lambda i: (0, 0)),
                  pl.BlockSpec((1, D), lambda i: (0, 0))],
        out_specs=pl.BlockSpec((BM, D), lambda i: (i, 0)),
        out_shape=jax.ShapeDtypeStruct((N, D), jnp.float32),
    )(p2, din_p, w_gc2, b_gc2.reshape(1, D))


def kernel(x, edge_index, W_fc, b_fc, b_gc1, W_gc2, b_gc2):
    src = edge_index[0].astype(jnp.int32)
    dst = edge_index[1].astype(jnp.int32)
    e = src.shape[0]
    cpt = -(-e // (NW * K))
    pad = NW * K * cpt - e
    fill = jnp.full((pad,), DUMMY, jnp.int32)
    src3 = jnp.concatenate([src, fill]).reshape(NW, cpt, K)
    dst3 = jnp.concatenate([dst, fill]).reshape(NW, cpt, K)
    xp = jnp.pad(x, ((0, NPAD - N), (0, 0)))
    zeros_d = jnp.zeros((NPAD, D), jnp.float32)
    zeros_l = jnp.zeros((NPAD, L), jnp.float32)
    ones_l = jnp.ones((K, L), jnp.float32)

    dout_p, din_p = _deg_partials(src3, dst3, ones_l, zeros_l, cpt)
    h0s = _tc1(xp, W_fc, b_fc, dout_p)
    p1 = _agg_partials(h0s, src3, dst3, zeros_d, cpt)
    h1s = _tc2(p1, dout_p, din_p, b_gc1)
    p2 = _agg_partials(h1s, src3, dst3, zeros_d, cpt)
    return _tc3(p2, din_p, W_gc2, b_gc2)


# speedup vs baseline: 3.9477x; 3.9477x over previous
import functools

import jax
import jax.numpy as jnp
from jax import lax
from jax.experimental import pallas as pl
from jax.experimental.pallas import tpu as pltpu
from jax.experimental.pallas import tpu_sc as plsc

N = 10000
D = 128
NC, NS, L = 2, 16, 16
NW = NC * NS
K = 128
NPAD = 10240
RPS = NPAD // NS
DUMMY = N
BM = 1024
GRID = NPAD // BM


def _sc_mesh():
    return plsc.VectorSubcoreMesh(
        core_axis_name="c", subcore_axis_name="s",
        num_cores=NC, num_subcores=NS)


def _deg_partials(idx3, ones0, zeros_d, cpt):
    @functools.partial(
        pl.kernel,
        out_type=jax.ShapeDtypeStruct((NC, NPAD, D), jnp.float32),
        mesh=_sc_mesh(),
        scratch_types=[
            pltpu.VMEM((cpt, K), jnp.int32),
            pltpu.VMEM((K, D), jnp.float32),
            pltpu.VMEM_SHARED((NPAD, D), jnp.float32),
        ],
    )
    def deg_k(idx_hbm, ones_hbm, z_hbm, deg_hbm, idx_v, ones_v, acc):
        cid = lax.axis_index("c")
        sid = lax.axis_index("s")
        wid = cid * NS + sid
        pltpu.sync_copy(idx_hbm.at[wid], idx_v)
        pltpu.sync_copy(ones_hbm, ones_v)
        r0 = sid * RPS
        pltpu.sync_copy(z_hbm, acc.at[pl.ds(r0, RPS)])
        plsc.subcore_barrier()

        def body(c, carry):
            pltpu.sync_copy(ones_v, acc.at[idx_v.at[c]], add=True)
            return carry

        lax.fori_loop(0, cpt, body, 0)
        plsc.subcore_barrier()
        pltpu.sync_copy(acc.at[pl.ds(r0, RPS)],
                        deg_hbm.at[cid, pl.ds(r0, RPS)])

    return deg_k(idx3, ones0, zeros_d)


def _agg_partials(h_pad, src3, dst3, zeros_d, cpt):
    @functools.partial(
        pl.kernel,
        out_type=jax.ShapeDtypeStruct((NC, NPAD, D), jnp.float32),
        mesh=_sc_mesh(),
        scratch_types=[
            pltpu.VMEM((cpt, K), jnp.int32),
            pltpu.VMEM((cpt, K), jnp.int32),
            pltpu.VMEM((K, D), jnp.float32),
            pltpu.VMEM_SHARED((NPAD, D), jnp.float32),
            pltpu.SemaphoreType.DMA,
        ],
    )
    def agg_k(h_hbm, src_hbm, dst_hbm, z_hbm, out_hbm,
              src_v, dst_v, rows_v, acc, sem):
        cid = lax.axis_index("c")
        sid = lax.axis_index("s")
        wid = cid * NS + sid
        pltpu.sync_copy(src_hbm.at[wid], src_v)
        pltpu.sync_copy(dst_hbm.at[wid], dst_v)
        r0 = sid * RPS
        pltpu.sync_copy(z_hbm, acc.at[pl.ds(r0, RPS)])
        plsc.subcore_barrier()

        def body(c, carry):
            pltpu.async_copy(h_hbm.at[src_v.at[c]], rows_v, sem).wait()
            pltpu.sync_copy(rows_v, acc.at[dst_v.at[c]], add=True)
            return carry

        lax.fori_loop(0, cpt, body, 0)
        plsc.subcore_barrier()
        pltpu.sync_copy(acc.at[pl.ds(r0, RPS)],
                        out_hbm.at[cid, pl.ds(r0, RPS)])

    return agg_k(h_pad, src3, dst3, zeros_d)


def _norm(pdeg):
    deg = (pdeg[0] + pdeg[1])[:, 0:1]
    return jnp.where(deg > 0, lax.rsqrt(deg), 0.0)


def _tc1_body(x_ref, w_ref, b_ref, po_ref, o_ref):
    ns = _norm(po_ref[...])
    h = jnp.dot(x_ref[...], w_ref[...], preferred_element_type=jnp.float32)
    o_ref[...] = (h + b_ref[...]) * ns


def _tc1(xp, w_fc, b_fc, dout_p):
    return pl.pallas_call(
        _tc1_body,
        grid=(GRID,),
        in_specs=[pl.BlockSpec((BM, D), lambda i: (i, 0)),
                  pl.BlockSpec((D, D), lambda i: (0, 0)),
                  pl.BlockSpec((1, D), lambda i: (0, 0)),
                  pl.BlockSpec((NC, BM, D), lambda i: (0, i, 0))],
        out_specs=pl.BlockSpec((BM, D), lambda i: (i, 0)),
        out_shape=jax.ShapeDtypeStruct((NPAD, D), jnp.float32),
    )(xp, w_fc, b_fc.reshape(1, D), dout_p)


def _tc2_body(a_ref, po_ref, pi_ref, b_ref, o_ref):
    agg = a_ref[0] + a_ref[1]
    ns = _norm(po_ref[...])
    nd = _norm(pi_ref[...])
    h = jnp.maximum(agg * nd + b_ref[...], 0.0)
    o_ref[...] = h * ns


def _tc2(p1, dout_p, din_p, b_gc1):
    return pl.pallas_call(
        _tc2_body,
        grid=(GRID,),
        in_specs=[pl.BlockSpec((NC, BM, D), lambda i: (0, i, 0)),
                  pl.BlockSpec((NC, BM, D), lambda i: (0, i, 0)),
                  pl.BlockSpec((NC, BM, D), lambda i: (0, i, 0)),
                  pl.BlockSpec((1, D), lambda i: (0, 0))],
        out_specs=pl.BlockSpec((BM, D), lambda i: (i, 0)),
        out_shape=jax.ShapeDtypeStruct((NPAD, D), jnp.float32),
    )(p1, dout_p, din_p, b_gc1.reshape(1, D))


def _tc3_body(a_ref, pi_ref, w_ref, b_ref, o_ref):
    agg = (a_ref[0] + a_ref[1]) * _norm(pi_ref[...])
    h = jnp.dot(agg, w_ref[...], preferred_element_type=jnp.float32)
    o_ref[...] = jnp.maximum(h + b_ref[...], 0.0)


def _tc3(p2, din_p, w_gc2, b_gc2):
    return pl.pallas_call(
        _tc3_body,
        grid=(GRID,),
        in_specs=[pl.BlockSpec((NC, BM, D), lambda i: (0, i, 0)),
                  pl.BlockSpec((NC, BM, D), lambda i: (0, i, 0)),
                  pl.BlockSpec((D, D), lambda i: (0, 0)),
                  pl.BlockSpec((1, D), lambda i: (0, 0))],
        out_specs=pl.BlockSpec((BM, D), lambda i: (i, 0)),
        out_shape=jax.ShapeDtypeStruct((N, D), jnp.float32),
    )(p2, din_p, w_gc2, b_gc2.reshape(1, D))


def kernel(x, edge_index, W_fc, b_fc, b_gc1, W_gc2, b_gc2):
    src = edge_index[0].astype(jnp.int32)
    dst = edge_index[1].astype(jnp.int32)
    e = src.shape[0]
    cpt = -(-e // (NW * K))
    pad = NW * K * cpt - e
    fill = jnp.full((pad,), DUMMY, jnp.int32)
    src3 = jnp.concatenate([src, fill]).reshape(NW, cpt, K)
    dst3 = jnp.concatenate([dst, fill]).reshape(NW, cpt, K)
    xp = jnp.pad(x, ((0, NPAD - N), (0, 0)))
    zeros_d = jnp.zeros((RPS, D), jnp.float32)
    ones0 = jnp.zeros((K, D), jnp.float32).at[:, 0].set(1.0)

    dout_p = _deg_partials(src3, ones0, zeros_d, cpt)
    din_p = _deg_partials(dst3, ones0, zeros_d, cpt)
    h0s = _tc1(xp, W_fc, b_fc, dout_p)
    p1 = _agg_partials(h0s, src3, dst3, zeros_d, cpt)
    h1s = _tc2(p1, dout_p, din_p, b_gc1)
    p2 = _agg_partials(h1s, src3, dst3, zeros_d, cpt)
    return _tc3(p2, din_p, W_gc2, b_gc2)


# speedup vs baseline: 3.9566x; 1.0022x over previous
import functools

import jax
import jax.numpy as jnp
from jax import lax
from jax.experimental import pallas as pl
from jax.experimental.pallas import tpu as pltpu
from jax.experimental.pallas import tpu_sc as plsc

N = 10000
D = 128
NC, NS, L = 2, 16, 16
NW = NC * NS
K = 128
NPAD = 10240
RPS = NPAD // NS
DUMMY = N
BM = 1024
GRID = NPAD // BM


def _sc_mesh():
    return plsc.VectorSubcoreMesh(
        core_axis_name="c", subcore_axis_name="s",
        num_cores=NC, num_subcores=NS)


def _deg_partials(idx3, ones0, zeros_d, cpt):
    @functools.partial(
        pl.kernel,
        out_type=jax.ShapeDtypeStruct((NC, NPAD, D), jnp.float32),
        mesh=_sc_mesh(),
        scratch_types=[
            pltpu.VMEM((cpt, K), jnp.int32),
            pltpu.VMEM((K, D), jnp.float32),
            pltpu.VMEM_SHARED((NPAD, D), jnp.float32),
            pltpu.SemaphoreType.DMA,
        ],
    )
    def deg_k(idx_hbm, ones_hbm, z_hbm, deg_hbm, idx_v, ones_v, acc, sem):
        cid = lax.axis_index("c")
        sid = lax.axis_index("s")
        wid = cid * NS + sid
        pltpu.sync_copy(idx_hbm.at[wid], idx_v)
        pltpu.sync_copy(ones_hbm, ones_v)
        r0 = sid * RPS
        pltpu.sync_copy(z_hbm, acc.at[pl.ds(r0, RPS)])
        plsc.subcore_barrier()

        def body(c, carry):
            pltpu.async_copy(ones_v, acc.at[idx_v.at[c]], sem, add=True)

            @pl.when(c > 0)
            def _():
                pltpu.make_async_copy(ones_v, acc.at[pl.ds(0, K)], sem).wait()

            return carry

        lax.fori_loop(0, cpt, body, 0)
        pltpu.make_async_copy(ones_v, acc.at[pl.ds(0, K)], sem).wait()
        plsc.subcore_barrier()
        pltpu.sync_copy(acc.at[pl.ds(r0, RPS)],
                        deg_hbm.at[cid, pl.ds(r0, RPS)])

    return deg_k(idx3, ones0, zeros_d)


def _agg_partials(h_pad, src3, dst3, zeros_d, cpt):
    @functools.partial(
        pl.kernel,
        out_type=jax.ShapeDtypeStruct((NC, NPAD, D), jnp.float32),
        mesh=_sc_mesh(),
        scratch_types=[
            pltpu.VMEM((cpt, K), jnp.int32),
            pltpu.VMEM((cpt, K), jnp.int32),
            pltpu.VMEM((K, D), jnp.float32),
            pltpu.VMEM_SHARED((NPAD, D), jnp.float32),
            pltpu.SemaphoreType.DMA,
        ],
    )
    def agg_k(h_hbm, src_hbm, dst_hbm, z_hbm, out_hbm,
              src_v, dst_v, rows_v, acc, sem):
        cid = lax.axis_index("c")
        sid = lax.axis_index("s")
        wid = cid * NS + sid
        pltpu.sync_copy(src_hbm.at[wid], src_v)
        pltpu.sync_copy(dst_hbm.at[wid], dst_v)
        r0 = sid * RPS
        pltpu.sync_copy(z_hbm, acc.at[pl.ds(r0, RPS)])
        plsc.subcore_barrier()

        def body(c, carry):
            pltpu.async_copy(h_hbm.at[src_v.at[c]], rows_v, sem).wait()
            pltpu.sync_copy(rows_v, acc.at[dst_v.at[c]], add=True)
            return carry

        lax.fori_loop(0, cpt, body, 0)
        plsc.subcore_barrier()
        pltpu.sync_copy(acc.at[pl.ds(r0, RPS)],
                        out_hbm.at[cid, pl.ds(r0, RPS)])

    return agg_k(h_pad, src3, dst3, zeros_d)


def _norm(pdeg):
    deg = (pdeg[0] + pdeg[1])[:, 0:1]
    return jnp.where(deg > 0, lax.rsqrt(deg), 0.0)


def _tc1_body(x_ref, w_ref, b_ref, po_ref, o_ref):
    ns = _norm(po_ref[...])
    h = jnp.dot(x_ref[...], w_ref[...], preferred_element_type=jnp.float32)
    o_ref[...] = (h + b_ref[...]) * ns


def _tc1(xp, w_fc, b_fc, dout_p):
    return pl.pallas_call(
        _tc1_body,
        grid=(GRID,),
        in_specs=[pl.BlockSpec((BM, D), lambda i: (i, 0)),
                  pl.BlockSpec((D, D), lambda i: (0, 0)),
                  pl.BlockSpec((1, D), lambda i: (0, 0)),
                  pl.BlockSpec((NC, BM, D), lambda i: (0, i, 0))],
        out_specs=pl.BlockSpec((BM, D), lambda i: (i, 0)),
        out_shape=jax.ShapeDtypeStruct((NPAD, D), jnp.float32),
    )(xp, w_fc, b_fc.reshape(1, D), dout_p)


def _tc2_body(a_ref, po_ref, pi_ref, b_ref, o_ref):
    agg = a_ref[0] + a_ref[1]
    ns = _norm(po_ref[...])
    nd = _norm(pi_ref[...])
    h = jnp.maximum(agg * nd + b_ref[...], 0.0)
    o_ref[...] = h * ns


def _tc2(p1, dout_p, din_p, b_gc1):
    return pl.pallas_call(
        _tc2_body,
        grid=(GRID,),
        in_specs=[pl.BlockSpec((NC, BM, D), lambda i: (0, i, 0)),
                  pl.BlockSpec((NC, BM, D), lambda i: (0, i, 0)),
                  pl.BlockSpec((NC, BM, D), lambda i: (0, i, 0)),
                  pl.BlockSpec((1, D), lambda i: (0, 0))],
        out_specs=pl.BlockSpec((BM, D), lambda i: (i, 0)),
        out_shape=jax.ShapeDtypeStruct((NPAD, D), jnp.float32),
    )(p1, dout_p, din_p, b_gc1.reshape(1, D))


def _tc3_body(a_ref, pi_ref, w_ref, b_ref, o_ref):
    agg = (a_ref[0] + a_ref[1]) * _norm(pi_ref[...])
    h = jnp.dot(agg, w_ref[...], preferred_element_type=jnp.float32)
    o_ref[...] = jnp.maximum(h + b_ref[...], 0.0)


def _tc3(p2, din_p, w_gc2, b_gc2):
    return pl.pallas_call(
        _tc3_body,
        grid=(GRID,),
        in_specs=[pl.BlockSpec((NC, BM, D), lambda i: (0, i, 0)),
                  pl.BlockSpec((NC, BM, D), lambda i: (0, i, 0)),
                  pl.BlockSpec((D, D), lambda i: (0, 0)),
                  pl.BlockSpec((1, D), lambda i: (0, 0))],
        out_specs=pl.BlockSpec((BM, D), lambda i: (i, 0)),
        out_shape=jax.ShapeDtypeStruct((N, D), jnp.float32),
    )(p2, din_p, w_gc2, b_gc2.reshape(1, D))


def kernel(x, edge_index, W_fc, b_fc, b_gc1, W_gc2, b_gc2):
    src = edge_index[0].astype(jnp.int32)
    dst = edge_index[1].astype(jnp.int32)
    e = src.shape[0]
    cpt = -(-e // (NW * K))
    pad = NW * K * cpt - e
    fill = jnp.full((pad,), DUMMY, jnp.int32)
    src3 = jnp.concatenate([src, fill]).reshape(NW, cpt, K)
    dst3 = jnp.concatenate([dst, fill]).reshape(NW, cpt, K)
    xp = jnp.pad(x, ((0, NPAD - N), (0, 0)))
    zeros_d = jnp.zeros((RPS, D), jnp.float32)
    ones0 = jnp.zeros((K, D), jnp.float32).at[:, 0].set(1.0)

    dout_p = _deg_partials(src3, ones0, zeros_d, cpt)
    din_p = _deg_partials(dst3, ones0, zeros_d, cpt)
    h0s = _tc1(xp, W_fc, b_fc, dout_p)
    p1 = _agg_partials(h0s, src3, dst3, zeros_d, cpt)
    h1s = _tc2(p1, dout_p, din_p, b_gc1)
    p2 = _agg_partials(h1s, src3, dst3, zeros_d, cpt)
    return _tc3(p2, din_p, W_gc2, b_gc2)
